# baseline (speedup 1.0000x reference)
import functools

import jax
import jax.numpy as jnp
from jax import lax
from jax.experimental import pallas as pl
from jax.experimental.pallas import tpu as pltpu
from jax.experimental.pallas import tpu_sc as plsc

_N = 10000
_E = 320000
_D = 128
_H = 128
_O = 64

_NC = 2
_NS = 16
_NW = _NC * _NS
_C = 128
_K1 = 160
_K2 = 80
_EP = _NS * _K1 * _C
_NP = 10240
_RP = _NP // _NS
_NB = 4
_W = 64


def _make_sc_pass(kchunks, split_half, with_deg):
  mesh = plsc.VectorSubcoreMesh(core_axis_name="c", subcore_axis_name="s")
  out_type = [jax.ShapeDtypeStruct((_NC, _NP, _W), jnp.float32)]
  scratch = [
      pltpu.VMEM((kchunks, _C), jnp.int32),
      pltpu.VMEM((kchunks, _C), jnp.int32),
  ]
  scratch += [pltpu.VMEM((_C, _W), jnp.float32) for _ in range(_NB)]
  scratch += [pltpu.SemaphoreType.DMA for _ in range(2 * _NB)]
  scratch.append(pltpu.VMEM_SHARED((_NP, _W), jnp.float32))
  if with_deg:
    out_type.append(jax.ShapeDtypeStruct((_NP,), jnp.float32))
    scratch += [
        pltpu.VMEM((_C,), jnp.float32),
        pltpu.VMEM((_RP,), jnp.float32),
        pltpu.VMEM_SHARED((_NP,), jnp.float32),
    ]
    scratch += [pltpu.SemaphoreType.DMA for _ in range(_NB)]

  def body(table, src_hbm, dst_hbm, *refs):
    out_agg = refs[0]
    refs = refs[1:]
    if with_deg:
      out_deg, refs = refs[0], refs[1:]
    src_v, dst_v = refs[0], refs[1]
    rows = refs[2:2 + _NB]
    gsem = refs[2 + _NB:2 + 2 * _NB]
    ssem = refs[2 + 2 * _NB:2 + 3 * _NB]
    acc_sh = refs[2 + 3 * _NB]
    if with_deg:
      ones_v, zdeg_v, deg_sh = refs[3 + 3 * _NB:6 + 3 * _NB]
      dsem = refs[6 + 3 * _NB:6 + 4 * _NB]
    c = lax.axis_index("c")
    s = lax.axis_index("s")
    base = s * _RP
    on_core0 = c == 0

    npack = _W // 16

    def zrow(t, carry):
      rows[0][t // npack, pl.ds((t % npack) * 16, 16)] = jnp.zeros(
          (16,), jnp.float32)
      return carry

    lax.fori_loop(0, _C * npack, zrow, 0)
    for k in range(_RP // _C):
      pltpu.sync_copy(rows[0], acc_sh.at[pl.ds(base + k * _C, _C)])

    if with_deg:
      @pl.when(on_core0)
      def _():
        def zdeg(t, carry):
          zdeg_v[pl.ds(t * 16, 16)] = jnp.zeros((16,), jnp.float32)
          return carry

        lax.fori_loop(0, _RP // 16, zdeg, 0)
        pltpu.sync_copy(zdeg_v, deg_sh.at[pl.ds(base, _RP)])

        def ones(t, carry):
          ones_v[pl.ds(t * 16, 16)] = jnp.ones((16,), jnp.float32)
          return carry

        lax.fori_loop(0, _C // 16, ones, 0)

    if split_half:
      pltpu.sync_copy(src_hbm.at[c, s], src_v)
      pltpu.sync_copy(dst_hbm.at[s], dst_v)
    else:
      wid = c * _NS + s
      pltpu.sync_copy(src_hbm.at[wid], src_v)
      pltpu.sync_copy(dst_hbm.at[wid], dst_v)

    plsc.subcore_barrier()

    for b in range(_NB):
      pltpu.async_copy(table.at[src_v.at[b]], rows[b], gsem[b])

    def group(g, carry):
      ch0 = g * _NB
      for b in range(_NB):
        ch = ch0 + b
        pltpu.make_async_copy(table.at[src_v.at[ch]], rows[b],
                              gsem[b]).wait()
        pltpu.async_copy(rows[b], acc_sh.at[dst_v.at[ch]], ssem[b],
                         add=True)
        if with_deg:
          @pl.when(on_core0)
          def _():
            pltpu.async_copy(ones_v, deg_sh.at[dst_v.at[ch]], dsem[b],
                             add=True)
      for b in range(_NB):
        ch = ch0 + b
        pltpu.make_async_copy(rows[b], acc_sh.at[dst_v.at[ch]],
                              ssem[b]).wait()
        if with_deg:
          @pl.when(on_core0)
          def _():
            pltpu.make_async_copy(ones_v, deg_sh.at[dst_v.at[ch]],
                                  dsem[b]).wait()
        nxt = jnp.minimum(ch + _NB, kchunks - 1)
        pltpu.async_copy(table.at[src_v.at[nxt]], rows[b], gsem[b])
      return carry

    lax.fori_loop(0, kchunks // _NB, group, 0)

    for b in range(_NB):
      pltpu.make_async_copy(table.at[src_v.at[kchunks - 1]], rows[b],
                            gsem[b]).wait()

    plsc.subcore_barrier()

    for k in range(_RP // _C):
      sl = pl.ds(base + k * _C, _C)
      pltpu.sync_copy(acc_sh.at[sl], out_agg.at[c, sl])
    if with_deg:
      @pl.when(on_core0)
      def _():
        pltpu.sync_copy(deg_sh.at[pl.ds(base, _RP)],
                        out_deg.at[pl.ds(base, _RP)])

  return pl.kernel(body, out_type=tuple(out_type), mesh=mesh,
                   scratch_types=scratch,
                   compiler_params=pltpu.CompilerParams(
                       use_tc_tiling_on_sc=False))


_sc_pass1 = _make_sc_pass(_K1, True, True)
_sc_pass2 = _make_sc_pass(_K2, False, False)

_BR = 1024
_GRID = _NP // _BR


def _tc_a_body(x_ref, ws1, wn1, sb, cb, agg, deg, wn2, h_ref, hw2_ref):
  d = jnp.maximum(deg[...], 1.0)
  hn = jnp.concatenate([agg[0], agg[1]], axis=-1) / d
  hl = jnp.dot(x_ref[...], ws1[...], preferred_element_type=jnp.float32)
  hl = hl + jnp.dot(hn, wn1[...], preferred_element_type=jnp.float32)
  h = jnp.maximum(hl * sb[...] + cb[...], 0.0)
  h_ref[...] = h
  hw2_ref[...] = jnp.dot(h, wn2[...], preferred_element_type=jnp.float32)


_tc_a = pl.pallas_call(
    _tc_a_body,
    grid=(_GRID,),
    in_specs=[
        pl.BlockSpec((_BR, _D), lambda i: (i, 0)),
        pl.BlockSpec((_D, _H), lambda i: (0, 0)),
        pl.BlockSpec((_D, _H), lambda i: (0, 0)),
        pl.BlockSpec((1, _H), lambda i: (0, 0)),
        pl.BlockSpec((1, _H), lambda i: (0, 0)),
        pl.BlockSpec((_NC, _BR, _W), lambda i: (0, i, 0)),
        pl.BlockSpec((_BR, 1), lambda i: (i, 0)),
        pl.BlockSpec((_H, _O), lambda i: (0, 0)),
    ],
    out_specs=[
        pl.BlockSpec((_BR, _H), lambda i: (i, 0)),
        pl.BlockSpec((_BR, _O), lambda i: (i, 0)),
    ],
    out_shape=[
        jax.ShapeDtypeStruct((_N, _H), jnp.float32),
        jax.ShapeDtypeStruct((_N, _O), jnp.float32),
    ],
)


def _tc_b_body(h_ref, ws2, agg2, deg, b2, out_ref):
  d = jnp.maximum(deg[...], 1.0)
  hn2 = (agg2[0] + agg2[1]) / d
  out_ref[...] = (
      jnp.dot(h_ref[...], ws2[...], preferred_element_type=jnp.float32)
      + hn2 + b2[...])


_tc_b = pl.pallas_call(
    _tc_b_body,
    grid=(_GRID,),
    in_specs=[
        pl.BlockSpec((_BR, _H), lambda i: (i, 0)),
        pl.BlockSpec((_H, _O), lambda i: (0, 0)),
        pl.BlockSpec((_NC, _BR, _O), lambda i: (0, i, 0)),
        pl.BlockSpec((_BR, 1), lambda i: (i, 0)),
        pl.BlockSpec((1, _O), lambda i: (0, 0)),
    ],
    out_specs=pl.BlockSpec((_BR, _O), lambda i: (i, 0)),
    out_shape=jax.ShapeDtypeStruct((_N, _O), jnp.float32),
)


def kernel(x, edge_index, W_self1, W_neigh1, b1, gamma1, beta1,
           W_self2, W_neigh2, b2):
  pad = _EP - _E
  srcp = jnp.concatenate([edge_index[0], jnp.zeros((pad,), jnp.int32)])
  dstp = jnp.concatenate([edge_index[1],
                          jnp.full((pad,), _NP - 1, jnp.int32)])
  s1 = (srcp.reshape(1, _NS, _K1, _C) * 2
        + jnp.arange(2, dtype=jnp.int32).reshape(2, 1, 1, 1))
  d1 = dstp.reshape(_NS, _K1, _C)
  s2 = srcp.reshape(_NW, _K2, _C)
  d2 = dstp.reshape(_NW, _K2, _C)

  aggx, deg = _sc_pass1(x.reshape(2 * _N, _W), s1, d1)
  deg2d = deg.reshape(_NP, 1)

  sb = (gamma1 * (1.0 / jnp.sqrt(1.0 + 1e-5))).reshape(1, _H)
  cb = (b1 * sb[0] + beta1).reshape(1, _H)

  h, hw2 = _tc_a(x, W_self1, W_neigh1, sb, cb, aggx, deg2d, W_neigh2)
  (agg2,) = _sc_pass2(hw2, s2, d2)
  out = _tc_b(h, W_self2, agg2, deg2d, b2.reshape(1, _O))
  return out
